# baseline (speedup 1.0000x reference)
import jax
import jax.numpy as jnp
from jax import lax
from jax.experimental import pallas as pl
from jax.experimental.pallas import tpu as pltpu
from jax.experimental.pallas import tpu_sc as plsc

NUM_RESIDUE_TYPE = 32
PAIR_DIM = 128
N_NODES = 10000
N_EDGES = 320000

NC, NS, L = 2, 16, 16
NW = NC * NS
BPW = N_EDGES // NW
CHUNK = 80
C2 = 2 * CHUNK
NCHUNK = BPW // CHUNK
VECS = CHUNK // L
K = 10
MACRO = NCHUNK // K
TAIL = NCHUNK - MACRO * K


def _body(edge_hbm, residue_hbm, weight_hbm, out_hbm, *scratch):
    didx = scratch[0:K]
    sdv = scratch[K:2 * K]
    rr = scratch[2 * K:3 * K]
    pair = scratch[3 * K:4 * K]
    rows = scratch[4 * K:5 * K]
    asem = scratch[5 * K:6 * K]
    wsem = scratch[6 * K:7 * K]
    sidx = scratch[7 * K]
    shw = scratch[7 * K + 1]
    shr = scratch[7 * K + 2]

    sid = lax.axis_index("s")
    wid = sid * NC + lax.axis_index("c")
    base = wid * BPW

    @pl.when(sid == 0)
    def _():
        pltpu.sync_copy(weight_hbm, shw)
        pltpu.sync_copy(residue_hbm, shr)

    lanes = lax.iota(jnp.int32, L)
    for j in range(VECS):
        sidx[pl.ds(j * L, L)] = lanes * 2 + j * (2 * L)
        sidx[pl.ds(CHUNK + j * L, L)] = lanes * 2 + 1 + j * (2 * L)

    plsc.subcore_barrier()

    def stage_in(ci, b):
        off = 2 * (base + ci * CHUNK)

        def dvec(j, c):
            o = j * L
            didx[b][pl.ds(o, L)] = sidx[pl.ds(o, L)] + off
            didx[b][pl.ds(CHUNK + o, L)] = sidx[pl.ds(CHUNK + o, L)] + off
            return c

        lax.fori_loop(0, VECS, dvec, 0, unroll=5)
        pltpu.async_copy(edge_hbm.at[didx[b]], sdv[b], asem[b])

    def fire_residue(b):
        pltpu.make_async_copy(edge_hbm.at[didx[b]], sdv[b], asem[b]).wait()
        pltpu.async_copy(shr.at[sdv[b]], rr[b], asem[b])

    def fire_weight(b, reuse):
        pltpu.make_async_copy(shr.at[sdv[b]], rr[b], asem[b]).wait()

        def vec(j, c):
            o = j * L
            pair[b][pl.ds(o, L)] = (
                (rr[b][pl.ds(o, L)] << 5) | rr[b][pl.ds(CHUNK + o, L)])
            return c

        lax.fori_loop(0, VECS, vec, 0, unroll=5)

        if reuse is not None:
            @pl.when(reuse)
            def _():
                pltpu.make_async_copy(
                    rows[b], out_hbm.at[pl.ds(base, CHUNK)], wsem[b]).wait()

        pltpu.async_copy(shw.at[pair[b]], rows[b], asem[b])

    def fire_out(ci, b):
        pltpu.make_async_copy(shw.at[pair[b]], rows[b], asem[b]).wait()
        pltpu.async_copy(rows[b], out_hbm.at[pl.ds(base + ci * CHUNK, CHUNK)],
                         wsem[b])

    def macro_body(m, carry):
        for b in range(K):
            stage_in(m * K + b, b)
        for b in range(K):
            fire_residue(b)
        for b in range(K):
            fire_weight(b, m > 0)
        for b in range(K):
            fire_out(m * K + b, b)
        return carry

    lax.fori_loop(0, MACRO, macro_body, 0)

    for t in range(TAIL):
        ci = MACRO * K + t
        stage_in(ci, t)
        fire_residue(t)
        fire_weight(t, jnp.bool_(True))
        fire_out(ci, t)

    for b in range(K):
        pltpu.make_async_copy(
            rows[b], out_hbm.at[pl.ds(base, CHUNK)], wsem[b]).wait()


@jax.jit
def kernel(residue, edge_index, weight):
    edge_flat = edge_index.reshape(-1).astype(jnp.int32)
    mesh = plsc.VectorSubcoreMesh(core_axis_name="c", subcore_axis_name="s",
                                  num_cores=NC, num_subcores=NS)
    scratch = (
        [pltpu.VMEM((C2,), jnp.int32) for _ in range(3 * K)]
        + [pltpu.VMEM((CHUNK,), jnp.int32) for _ in range(K)]
        + [pltpu.VMEM((CHUNK, PAIR_DIM), jnp.float32) for _ in range(K)]
        + [pltpu.SemaphoreType.DMA for _ in range(2 * K)]
        + [pltpu.VMEM((C2,), jnp.int32),
           pltpu.VMEM_SHARED((NUM_RESIDUE_TYPE * NUM_RESIDUE_TYPE, PAIR_DIM),
                             jnp.float32),
           pltpu.VMEM_SHARED((N_NODES,), jnp.int32)]
    )
    fn = pl.kernel(
        _body,
        out_type=jax.ShapeDtypeStruct((N_EDGES, PAIR_DIM), jnp.float32),
        mesh=mesh,
        scratch_types=scratch,
    )
    return fn(edge_flat, residue.astype(jnp.int32), weight)


# speedup vs baseline: 2.4518x; 2.4518x over previous
import jax
import jax.numpy as jnp
from jax import lax
from jax.experimental import pallas as pl
from jax.experimental.pallas import tpu as pltpu
from jax.experimental.pallas import tpu_sc as plsc

NUM_RESIDUE_TYPE = 32
PAIR_DIM = 128
N_NODES = 10000
N_EDGES = 320000

NC, NS, L = 2, 16, 16
NW = NC * NS
BPW = N_EDGES // NW
CHUNK = 80
C2 = 2 * CHUNK
NCHUNK = BPW // CHUNK
NCHUNK_ALL = N_EDGES // CHUNK
VECS = CHUNK // L
K = 11
MACRO = NCHUNK // K
TAIL = NCHUNK - MACRO * K


def _body(edge_hbm, residue_hbm, weight_hbm, out_hbm, *scratch):
    ev = scratch[0:K]
    rr = scratch[K:2 * K]
    pair = scratch[2 * K:3 * K]
    rows = scratch[3 * K:4 * K]
    asem = scratch[4 * K:5 * K]
    wsem = scratch[5 * K:6 * K]
    shw = scratch[6 * K]
    shr = scratch[6 * K + 1]

    sid = lax.axis_index("s")
    wid = sid * NC + lax.axis_index("c")
    base = wid * BPW
    gbase = wid * NCHUNK

    @pl.when(sid == 0)
    def _():
        pltpu.sync_copy(weight_hbm, shw)
        pltpu.sync_copy(residue_hbm, shr)

    plsc.subcore_barrier()

    def stage_in(ci, b):
        off = (gbase + ci) * C2
        pltpu.async_copy(edge_hbm.at[pl.ds(off, C2)], ev[b], asem[b])

    def fire_residue(b):
        pltpu.make_async_copy(edge_hbm.at[pl.ds(0, C2)], ev[b],
                              asem[b]).wait()
        pltpu.async_copy(shr.at[ev[b]], rr[b], asem[b])

    def fire_weight(b, reuse):
        pltpu.make_async_copy(shr.at[ev[b]], rr[b], asem[b]).wait()

        def vec(j, c):
            o = j * L
            pair[b][pl.ds(o, L)] = (
                (rr[b][pl.ds(o, L)] << 5) | rr[b][pl.ds(CHUNK + o, L)])
            return c

        lax.fori_loop(0, VECS, vec, 0, unroll=5)

        if reuse is not None:
            @pl.when(reuse)
            def _():
                pltpu.make_async_copy(
                    rows[b], out_hbm.at[pl.ds(base, CHUNK)], wsem[b]).wait()

        pltpu.async_copy(shw.at[pair[b]], rows[b], asem[b])

    def fire_out(ci, b):
        pltpu.make_async_copy(shw.at[pair[b]], rows[b], asem[b]).wait()
        pltpu.async_copy(rows[b], out_hbm.at[pl.ds(base + ci * CHUNK, CHUNK)],
                         wsem[b])

    def macro_body(m, carry):
        for b in range(K):
            stage_in(m * K + b, b)
        for b in range(K):
            fire_residue(b)
        for b in range(K):
            fire_weight(b, m > 0)
        for b in range(K):
            fire_out(m * K + b, b)
        return carry

    lax.fori_loop(0, MACRO, macro_body, 0)

    for t in range(TAIL):
        ci = MACRO * K + t
        stage_in(ci, t)
        fire_residue(t)
        fire_weight(t, jnp.bool_(True))
        fire_out(ci, t)

    for b in range(K):
        pltpu.make_async_copy(
            rows[b], out_hbm.at[pl.ds(base, CHUNK)], wsem[b]).wait()


@jax.jit
def kernel(residue, edge_index, weight):
    edge_re = (edge_index.astype(jnp.int32)
               .reshape(NCHUNK_ALL, CHUNK, 2)
               .transpose(0, 2, 1)
               .reshape(-1))
    mesh = plsc.VectorSubcoreMesh(core_axis_name="c", subcore_axis_name="s",
                                  num_cores=NC, num_subcores=NS)
    scratch = (
        [pltpu.VMEM((C2,), jnp.int32) for _ in range(2 * K)]
        + [pltpu.VMEM((CHUNK,), jnp.int32) for _ in range(K)]
        + [pltpu.VMEM((CHUNK, PAIR_DIM), jnp.float32) for _ in range(K)]
        + [pltpu.SemaphoreType.DMA for _ in range(2 * K)]
        + [pltpu.VMEM_SHARED((NUM_RESIDUE_TYPE * NUM_RESIDUE_TYPE, PAIR_DIM),
                             jnp.float32),
           pltpu.VMEM_SHARED((N_NODES,), jnp.int32)]
    )
    fn = pl.kernel(
        _body,
        out_type=jax.ShapeDtypeStruct((N_EDGES, PAIR_DIM), jnp.float32),
        mesh=mesh,
        scratch_types=scratch,
    )
    return fn(edge_re, residue.astype(jnp.int32), weight)
